# baseline (speedup 1.0000x reference)
import functools

import jax
import jax.numpy as jnp
from jax.experimental import pallas as pl
from jax.experimental.pallas import tpu as pltpu

TOP_K_RATIO = 0.2
CHUNKS = [128, 128, 256] + [512] * 14 + [256, 128, 128]
NBUF = 3
CN_MAX = max(CHUNKS)


def _select_kernel(x_hbm, mask_ref, buf, norms_ref, sems, *, top_k):
    n_ck = len(CHUNKS)
    offs = [0]
    for cn in CHUNKS:
        offs.append(offs[-1] + cn)

    def copy(i):
        cn, off, slot = CHUNKS[i], offs[i], i % NBUF
        return pltpu.make_async_copy(
            x_hbm.at[:, pl.ds(off, cn), :],
            buf.at[slot, :, pl.ds(0, cn), :],
            sems.at[slot])

    for i in range(NBUF):
        copy(i).start()
    for i in range(n_ck):
        copy(i).wait()
        x = buf[i % NBUF, :, 0:CHUNKS[i], :]
        norms_ref[:, offs[i]:offs[i] + CHUNKS[i]] = jnp.sqrt(
            jnp.sum(x * x, axis=2))
        if i + NBUF < n_ck:
            copy(i + NBUF).start()

    v = norms_ref[...]
    b_dim, n_dim = v.shape
    col = jax.lax.broadcasted_iota(jnp.int32, (b_dim, n_dim), 1)
    v = jnp.where(col == 0, 0.0, v)
    bv = jax.lax.bitcast_convert_type(v, jnp.int32)

    bv3 = bv.reshape(b_dim, 1, n_dim)
    col3 = col.reshape(b_dim, 1, n_dim)
    jv = (jax.lax.broadcasted_iota(jnp.int32, (1, 8, 1), 1) + 1)

    def probes(lo, hi):
        s = hi - lo + 1
        return lo + (s // 9) * jv + ((s % 9) * jv) // 9

    def val_step(_, lohi):
        lo, hi = lohi
        mid = probes(lo, hi)
        cnt = jnp.sum((bv3 >= mid).astype(jnp.int32), axis=2,
                      keepdims=True)
        ge = cnt >= top_k
        new_lo = jnp.max(jnp.where(ge, mid, lo), axis=1, keepdims=True)
        new_hi = jnp.min(jnp.where(ge, hi, mid - 1), axis=1,
                         keepdims=True)
        return new_lo, new_hi

    lo0 = jnp.zeros((b_dim, 1, 1), jnp.int32)
    hi0 = jnp.full((b_dim, 1, 1), 0x7F800000, jnp.int32)
    t3, _ = jax.lax.fori_loop(0, 10, val_step, (lo0, hi0))
    t_bits = t3.reshape(b_dim, 1)

    gt = bv > t_bits
    eq = bv == t_bits
    n_gt = jnp.sum(gt.astype(jnp.int32), axis=1, keepdims=True)
    n_eq = jnp.sum(eq.astype(jnp.int32), axis=1, keepdims=True)
    r = top_k - n_gt

    eq3 = eq.reshape(b_dim, 1, n_dim)
    r3 = r.reshape(b_dim, 1, 1)

    def idx_search(_):
        def idx_step(_, lohi):
            lo, hi = lohi
            mid = probes(lo, hi)
            cnt = jnp.sum((eq3 & (col3 <= mid)).astype(jnp.int32),
                          axis=2, keepdims=True)
            ok = cnt >= r3
            new_hi = jnp.min(jnp.where(ok, mid, hi), axis=1,
                             keepdims=True)
            new_lo = jnp.max(jnp.where(ok, lo, mid + 1), axis=1,
                             keepdims=True)
            return new_lo, new_hi

        ilo0 = jnp.zeros((b_dim, 1, 1), jnp.int32)
        ihi0 = jnp.full((b_dim, 1, 1), n_dim - 1, jnp.int32)
        i3, _ = jax.lax.fori_loop(0, 5, idx_step, (ilo0, ihi0))
        return i3.reshape(b_dim, 1)

    i_sel = jax.lax.cond(
        jnp.all(r == n_eq),
        lambda _: jnp.full((b_dim, 1), n_dim - 1, jnp.int32),
        idx_search,
        operand=0,
    )

    mask = gt | (eq & (col <= i_sel)) | (col == 0)
    mask_ref[...] = mask.astype(jnp.int8)


def kernel(img_tokens):
    B, N, D = img_tokens.shape
    top_k = max(1, int(N * TOP_K_RATIO))
    mask_i8 = pl.pallas_call(
        functools.partial(_select_kernel, top_k=top_k),
        in_specs=[pl.BlockSpec(memory_space=pl.ANY)],
        out_specs=pl.BlockSpec((B, N), lambda: (0, 0)),
        out_shape=jax.ShapeDtypeStruct((B, N), jnp.int8),
        scratch_shapes=[
            pltpu.VMEM((NBUF, B, CN_MAX, D), jnp.float32),
            pltpu.VMEM((B, N), jnp.float32),
            pltpu.SemaphoreType.DMA((NBUF,)),
        ],
    )(img_tokens)
    return mask_i8.astype(bool)


# speedup vs baseline: 1.0603x; 1.0603x over previous
import functools

import jax
import jax.numpy as jnp
from jax.experimental import pallas as pl
from jax.experimental.pallas import tpu as pltpu

TOP_K_RATIO = 0.2
CN = 512


def _select_kernel(x_ref, mask_ref, norms_ref, *, n_chunks, top_k):
    c = pl.program_id(0)

    x = x_ref[...]
    norms_ref[:, pl.ds(c * CN, CN)] = jnp.sqrt(jnp.sum(x * x, axis=2))

    @pl.when(c == n_chunks - 1)
    def _phase_b():
        v = norms_ref[...]
        b_dim, n_dim = v.shape
        col = jax.lax.broadcasted_iota(jnp.int32, (b_dim, n_dim), 1)
        v = jnp.where(col == 0, 0.0, v)
        bv = jax.lax.bitcast_convert_type(v, jnp.int32)

        bv3 = bv.reshape(b_dim, 1, n_dim)
        col3 = col.reshape(b_dim, 1, n_dim)
        jv = (jax.lax.broadcasted_iota(jnp.int32, (1, 8, 1), 1) + 1)

        def probes(lo, hi):
            s = hi - lo + 1
            return lo + (s // 9) * jv + ((s % 9) * jv) // 9

        def val_step(_, lohi):
            lo, hi = lohi
            mid = probes(lo, hi)
            cnt = jnp.sum((bv3 >= mid).astype(jnp.int32), axis=2,
                          keepdims=True)
            ge = cnt >= top_k
            new_lo = jnp.max(jnp.where(ge, mid, lo), axis=1, keepdims=True)
            new_hi = jnp.min(jnp.where(ge, hi, mid - 1), axis=1,
                             keepdims=True)
            return new_lo, new_hi

        lo0 = jnp.zeros((b_dim, 1, 1), jnp.int32)
        hi0 = jnp.full((b_dim, 1, 1), 0x7F800000, jnp.int32)
        t3, _ = jax.lax.fori_loop(0, 10, val_step, (lo0, hi0))
        t_bits = t3.reshape(b_dim, 1)

        gt = bv > t_bits
        eq = bv == t_bits
        n_gt = jnp.sum(gt.astype(jnp.int32), axis=1, keepdims=True)
        n_eq = jnp.sum(eq.astype(jnp.int32), axis=1, keepdims=True)
        r = top_k - n_gt

        eq3 = eq.reshape(b_dim, 1, n_dim)
        r3 = r.reshape(b_dim, 1, 1)

        def idx_search(_):
            def idx_step(_, lohi):
                lo, hi = lohi
                mid = probes(lo, hi)
                cnt = jnp.sum((eq3 & (col3 <= mid)).astype(jnp.int32),
                              axis=2, keepdims=True)
                ok = cnt >= r3
                new_hi = jnp.min(jnp.where(ok, mid, hi), axis=1,
                                 keepdims=True)
                new_lo = jnp.max(jnp.where(ok, lo, mid + 1), axis=1,
                                 keepdims=True)
                return new_lo, new_hi

            ilo0 = jnp.zeros((b_dim, 1, 1), jnp.int32)
            ihi0 = jnp.full((b_dim, 1, 1), n_dim - 1, jnp.int32)
            i3, _ = jax.lax.fori_loop(0, 5, idx_step, (ilo0, ihi0))
            return i3.reshape(b_dim, 1)

        i_sel = jax.lax.cond(
            jnp.all(r == n_eq),
            lambda _: jnp.full((b_dim, 1), n_dim - 1, jnp.int32),
            idx_search,
            operand=0,
        )

        mask = gt | (eq & (col <= i_sel)) | (col == 0)
        mask_ref[...] = mask.astype(jnp.int8)


def kernel(img_tokens):
    B, N, D = img_tokens.shape
    top_k = max(1, int(N * TOP_K_RATIO))
    n_chunks = N // CN
    grid = (n_chunks,)
    mask_i8 = pl.pallas_call(
        functools.partial(_select_kernel, n_chunks=n_chunks, top_k=top_k),
        grid=grid,
        in_specs=[pl.BlockSpec((B, CN, D), lambda c: (0, c, 0))],
        out_specs=pl.BlockSpec((B, N), lambda c: (0, 0)),
        out_shape=jax.ShapeDtypeStruct((B, N), jnp.int8),
        scratch_shapes=[pltpu.VMEM((B, N), jnp.float32)],
    )(img_tokens)
    return mask_i8.astype(bool)
